# baseline (speedup 1.0000x reference)
import jax
import jax.numpy as jnp
from jax import lax
from jax.experimental import pallas as pl
from jax.experimental.pallas import tpu as pltpu

_THRESH = 0.5
_ALPHA = 0.25
_PT = 1024
_NPAD = 32


def _iou_tile(priors_ref, boxesT_ref):
    pr = priors_ref[...]
    pcx = pr[:, 0:1]
    pcy = pr[:, 1:2]
    pw = pr[:, 2:3]
    ph = pr[:, 3:4]
    px0 = pcx - pw * 0.5
    py0 = pcy - ph * 0.5
    px1 = pcx + pw * 0.5
    py1 = pcy + ph * 0.5
    bt = boxesT_ref[0]
    bx0 = bt[0:1, :]
    by0 = bt[1:2, :]
    bx1 = bt[2:3, :]
    by1 = bt[3:4, :]
    ix0 = jnp.maximum(px0, bx0)
    iy0 = jnp.maximum(py0, by0)
    ix1 = jnp.minimum(px1, bx1)
    iy1 = jnp.minimum(py1, by1)
    inter = jnp.maximum(ix1 - ix0, 0.0) * jnp.maximum(iy1 - iy0, 0.0)
    pa = (px1 - px0) * (py1 - py0)
    ba = (bx1 - bx0) * (by1 - by0)
    ov = inter / (pa + ba - inter)
    return ov, pcx, pcy, pw, ph, bx0, by0, bx1, by1


def _match_kernel(nP, nPT, nobj, priors_ref, boxesT_ref, out_ref, vacc, iacc):
    pt = pl.program_id(1)
    ov = _iou_tile(priors_ref, boxesT_ref)[0]
    sidx = lax.broadcasted_iota(jnp.int32, ov.shape, 0)
    gp0 = pt * _PT
    valid = (sidx + gp0) < nP
    ovm = jnp.where(valid, ov, -1.0)
    tmax = jnp.max(ovm, axis=0, keepdims=True)
    tidx = jnp.min(jnp.where(ovm == tmax, sidx, jnp.int32(2 ** 30)),
                   axis=0, keepdims=True) + gp0

    @pl.when(pt == 0)
    def _():
        vacc[...] = tmax
        iacc[...] = tidx

    @pl.when(pt > 0)
    def _():
        better = tmax > vacc[...]
        iacc[...] = jnp.where(better, tidx, iacc[...])
        vacc[...] = jnp.where(better, tmax, vacc[...])

    @pl.when(pt == nPT - 1)
    def _():
        lane = lax.broadcasted_iota(jnp.int32, (1, _NPAD), 1)
        out_ref[0] = jnp.where(lane < nobj, iacc[...], -1)


def _loss_kernel(nP, nPT, nB, nobj, scores_ref, locs_ref, priors_ref,
                 boxesT_ref, labels_ref, pfe_ref, out_ref,
                 a_fl, a_m, a_d, a_p):
    b = pl.program_id(0)
    pt = pl.program_id(1)

    @pl.when((b == 0) & (pt == 0))
    def _():
        z = jnp.zeros((1, 1), jnp.float32)
        a_fl[...] = z
        a_m[...] = z
        a_d[...] = z
        a_p[...] = z

    ov, pcx, pcy, pw, ph, bx0, by0, bx1, by1 = _iou_tile(priors_ref, boxesT_ref)
    lane = lax.broadcasted_iota(jnp.int32, ov.shape, 1)
    ovm = jnp.where(lane < nobj, ov, -1e30)
    ovmax = jnp.max(ovm, axis=1, keepdims=True)
    obj = jnp.min(jnp.where(ovm == ovmax, lane, jnp.int32(64)),
                  axis=1, keepdims=True)

    sidx = lax.broadcasted_iota(jnp.int32, (ov.shape[0], 1), 0)
    gp = sidx + pt * _PT
    valid = gp < nP

    match = gp == pfe_ref[0]
    mo = jnp.max(jnp.where(match, lane, -1), axis=1, keepdims=True)
    hit = mo >= 0
    obj = jnp.where(hit, mo, obj)
    ovmax = jnp.where(hit, 1.0, ovmax)

    onehot = lane == obj
    lab = jnp.sum(jnp.where(onehot, labels_ref[0], 0.0), axis=1, keepdims=True)
    gx0 = jnp.sum(jnp.where(onehot, bx0, 0.0), axis=1, keepdims=True)
    gy0 = jnp.sum(jnp.where(onehot, by0, 0.0), axis=1, keepdims=True)
    gx1 = jnp.sum(jnp.where(onehot, bx1, 0.0), axis=1, keepdims=True)
    gy1 = jnp.sum(jnp.where(onehot, by1, 0.0), axis=1, keepdims=True)

    pos = (ovmax >= _THRESH) & valid
    neg = (ovmax < _THRESH - 0.1) & valid
    msk = pos | neg
    tcls = jnp.where(pos, lab, 0.0)

    x = scores_ref[0]
    mx = jnp.max(x, axis=1, keepdims=True)
    s = x - mx
    lse = jnp.log(jnp.sum(jnp.exp(s), axis=1, keepdims=True))
    cl = lax.broadcasted_iota(jnp.int32, x.shape, 1)
    st = jnp.sum(jnp.where(cl == tcls.astype(jnp.int32), s, 0.0),
                 axis=1, keepdims=True)
    logpt = st - lse
    ptv = jnp.exp(logpt)
    al = jnp.where(pos, _ALPHA, 1.0 - _ALPHA)
    om = 1.0 - ptv
    fl = -al * om * om * logpt

    bcx = (gx0 + gx1) * 0.5
    bcy = (gy0 + gy1) * 0.5
    bw = gx1 - gx0
    bh = gy1 - gy0
    t0 = (bcx - pcx) / (pw * 0.1)
    t1 = (bcy - pcy) / (ph * 0.1)
    t2 = jnp.log(bw / pw) * 5.0
    t3 = jnp.log(bh / ph) * 5.0
    lo = locs_ref[0]
    d = (jnp.abs(lo[:, 0:1] - t0) + jnp.abs(lo[:, 1:2] - t1)
         + jnp.abs(lo[:, 2:3] - t2) + jnp.abs(lo[:, 3:4] - t3))

    a_fl[...] += jnp.sum(jnp.where(msk, fl, 0.0)).reshape(1, 1)
    a_m[...] += jnp.sum(jnp.where(msk, 1.0, 0.0)).reshape(1, 1)
    a_d[...] += jnp.sum(jnp.where(pos, d, 0.0)).reshape(1, 1)
    a_p[...] += jnp.sum(jnp.where(pos, 1.0, 0.0)).reshape(1, 1)

    @pl.when((b == nB - 1) & (pt == nPT - 1))
    def _():
        out_ref[...] = (a_fl[...] / jnp.maximum(a_m[...], 1.0)
                        + a_d[...] / jnp.maximum(a_p[...] * 4.0, 1.0))


def kernel(predicted_locs, predicted_scores, boxes, priors_cxcy, labels):
    B, P, C = predicted_scores.shape
    NOBJ = boxes.shape[1]
    PT = (P + _PT - 1) // _PT

    boxesT = jnp.pad(boxes, ((0, 0), (0, _NPAD - NOBJ), (0, 0))
                     ).transpose(0, 2, 1)
    labelsf = jnp.pad(labels.astype(jnp.float32),
                      ((0, 0), (0, _NPAD - NOBJ)))[:, None, :]

    pfe = pl.pallas_call(
        lambda *refs: _match_kernel(P, PT, NOBJ, *refs),
        grid=(B, PT),
        in_specs=[
            pl.BlockSpec((_PT, 4), lambda b, pt: (pt, 0)),
            pl.BlockSpec((1, 4, _NPAD), lambda b, pt: (b, 0, 0)),
        ],
        out_specs=pl.BlockSpec((1, 1, _NPAD), lambda b, pt: (b, 0, 0)),
        out_shape=jax.ShapeDtypeStruct((B, 1, _NPAD), jnp.int32),
        scratch_shapes=[
            pltpu.VMEM((1, _NPAD), jnp.float32),
            pltpu.VMEM((1, _NPAD), jnp.int32),
        ],
    )(priors_cxcy, boxesT)

    loss = pl.pallas_call(
        lambda *refs: _loss_kernel(P, PT, B, NOBJ, *refs),
        grid=(B, PT),
        in_specs=[
            pl.BlockSpec((1, _PT, C), lambda b, pt: (b, pt, 0)),
            pl.BlockSpec((1, _PT, 4), lambda b, pt: (b, pt, 0)),
            pl.BlockSpec((_PT, 4), lambda b, pt: (pt, 0)),
            pl.BlockSpec((1, 4, _NPAD), lambda b, pt: (b, 0, 0)),
            pl.BlockSpec((1, 1, _NPAD), lambda b, pt: (b, 0, 0)),
            pl.BlockSpec((1, 1, _NPAD), lambda b, pt: (b, 0, 0)),
        ],
        out_specs=pl.BlockSpec((1, 1), lambda b, pt: (0, 0)),
        out_shape=jax.ShapeDtypeStruct((1, 1), jnp.float32),
        scratch_shapes=[
            pltpu.VMEM((1, 1), jnp.float32),
            pltpu.VMEM((1, 1), jnp.float32),
            pltpu.VMEM((1, 1), jnp.float32),
            pltpu.VMEM((1, 1), jnp.float32),
        ],
    )(predicted_scores, predicted_locs, priors_cxcy, boxesT, labelsf, pfe)

    return loss[0, 0]


# speedup vs baseline: 2.3365x; 2.3365x over previous
import jax
import jax.numpy as jnp
from jax import lax
from jax.experimental import pallas as pl
from jax.experimental.pallas import tpu as pltpu

_THRESH = 0.5
_ALPHA = 0.25
_L = 2048
_PT = 2048
_NPAD = 32


def _iou_lanes(priors_ref, boxes_ref):
    pr = priors_ref[...]
    pcx = pr[0:1, :]
    pcy = pr[1:2, :]
    pw = pr[2:3, :]
    ph = pr[3:4, :]
    px0 = pcx - pw * 0.5
    py0 = pcy - ph * 0.5
    px1 = pcx + pw * 0.5
    py1 = pcy + ph * 0.5
    bo = boxes_ref[0]
    bx0 = bo[:, 0:1]
    by0 = bo[:, 1:2]
    bx1 = bo[:, 2:3]
    by1 = bo[:, 3:4]
    ix0 = jnp.maximum(px0, bx0)
    iy0 = jnp.maximum(py0, by0)
    ix1 = jnp.minimum(px1, bx1)
    iy1 = jnp.minimum(py1, by1)
    inter = jnp.maximum(ix1 - ix0, 0.0) * jnp.maximum(iy1 - iy0, 0.0)
    pa = (px1 - px0) * (py1 - py0)
    ba = (bx1 - bx0) * (by1 - by0)
    ov = inter / (pa + ba - inter)
    return ov, pcx, pcy, pw, ph, bx0, by0, bx1, by1


def _match_kernel(nP, nPL, nobj, priors_ref, boxes_ref, out_ref, vacc, iacc):
    l = pl.program_id(1)
    ov = _iou_lanes(priors_ref, boxes_ref)[0]
    glob = lax.broadcasted_iota(jnp.int32, ov.shape, 1) + l * _L
    ovm = jnp.where(glob < nP, ov, -1.0)
    rmax = jnp.max(ovm, axis=1, keepdims=True)
    ridx = jnp.min(jnp.where(ovm == rmax, glob, jnp.int32(2 ** 30)),
                   axis=1, keepdims=True)

    @pl.when(l == 0)
    def _():
        vacc[...] = rmax
        iacc[...] = ridx

    @pl.when(l > 0)
    def _():
        better = rmax > vacc[...]
        iacc[...] = jnp.where(better, ridx, iacc[...])
        vacc[...] = jnp.where(better, rmax, vacc[...])

    @pl.when(l == nPL - 1)
    def _():
        soi = lax.broadcasted_iota(jnp.int32, (_NPAD, 1), 0)
        out_ref[0] = jnp.where(soi < nobj, iacc[...], -1)


def _prep_kernel(nP, nPL, nB, nobj, priors_ref, boxes_ref, labels_ref,
                 pfe_ref, locsT_ref, tcls_ref, coef_ref,
                 sm_ref, sp_ref, sd_ref, a_m, a_p, a_d):
    b = pl.program_id(0)
    l = pl.program_id(1)

    @pl.when((b == 0) & (l == 0))
    def _():
        z = jnp.zeros((1, 1), jnp.float32)
        a_m[...] = z
        a_p[...] = z
        a_d[...] = z

    ov, pcx, pcy, pw, ph, bx0, by0, bx1, by1 = _iou_lanes(priors_ref, boxes_ref)
    soi = lax.broadcasted_iota(jnp.int32, ov.shape, 0)
    ovm = jnp.where(soi < nobj, ov, -1e30)
    ovmax = jnp.max(ovm, axis=0, keepdims=True)
    obj = jnp.min(jnp.where(ovm == ovmax, soi, jnp.int32(64)),
                  axis=0, keepdims=True)

    lidx = lax.broadcasted_iota(jnp.int32, (1, _L), 1) + l * _L
    validp = lidx < nP

    matchm = pfe_ref[0] == lidx
    mo = jnp.max(jnp.where(matchm, soi, -1), axis=0, keepdims=True)
    hit = mo >= 0
    obj = jnp.where(hit, mo, obj)
    ovmax = jnp.where(hit, 1.0, ovmax)

    onehot = soi == obj
    lab = jnp.sum(jnp.where(onehot, labels_ref[0], 0.0), axis=0, keepdims=True)
    gx0 = jnp.sum(jnp.where(onehot, bx0, 0.0), axis=0, keepdims=True)
    gy0 = jnp.sum(jnp.where(onehot, by0, 0.0), axis=0, keepdims=True)
    gx1 = jnp.sum(jnp.where(onehot, bx1, 0.0), axis=0, keepdims=True)
    gy1 = jnp.sum(jnp.where(onehot, by1, 0.0), axis=0, keepdims=True)

    pos = (ovmax >= _THRESH) & validp
    neg = (ovmax < _THRESH - 0.1) & validp
    msk = pos | neg
    tcls_ref[0] = jnp.where(pos, lab, 0.0).astype(jnp.int32)
    coef_ref[0] = jnp.where(msk, jnp.where(pos, -_ALPHA, _ALPHA - 1.0), 0.0)

    bcx = (gx0 + gx1) * 0.5
    bcy = (gy0 + gy1) * 0.5
    bw = gx1 - gx0
    bh = gy1 - gy0
    t0 = (bcx - pcx) / (pw * 0.1)
    t1 = (bcy - pcy) / (ph * 0.1)
    t2 = jnp.log(bw / pw) * 5.0
    t3 = jnp.log(bh / ph) * 5.0
    lt = locsT_ref[0]
    d = (jnp.abs(lt[0:1, :] - t0) + jnp.abs(lt[1:2, :] - t1)
         + jnp.abs(lt[2:3, :] - t2) + jnp.abs(lt[3:4, :] - t3))

    a_m[...] += jnp.sum(jnp.where(msk, 1.0, 0.0)).reshape(1, 1)
    a_p[...] += jnp.sum(jnp.where(pos, 1.0, 0.0)).reshape(1, 1)
    a_d[...] += jnp.sum(jnp.where(pos, d, 0.0)).reshape(1, 1)

    @pl.when((b == nB - 1) & (l == nPL - 1))
    def _():
        sm_ref[...] = a_m[...]
        sp_ref[...] = a_p[...]
        sd_ref[...] = a_d[...]


def _loss_kernel(nPT, nB, scores_ref, tcls_ref, coef_ref,
                 sm_ref, sp_ref, sd_ref, out_ref, a_fl):
    b = pl.program_id(0)
    pt = pl.program_id(1)

    @pl.when((b == 0) & (pt == 0))
    def _():
        a_fl[...] = jnp.zeros((1, 1), jnp.float32)

    x = scores_ref[0]
    tci = tcls_ref[0]
    coef = coef_ref[0]
    mx = jnp.max(x, axis=1, keepdims=True)
    s = x - mx
    lse = jnp.log(jnp.sum(jnp.exp(s), axis=1, keepdims=True))
    cl = lax.broadcasted_iota(jnp.int32, x.shape, 1)
    st = jnp.sum(jnp.where(cl == tci, s, 0.0), axis=1, keepdims=True)
    logpt = st - lse
    om = 1.0 - jnp.exp(logpt)
    f = coef * (om * om) * logpt
    f = jnp.where(coef < 0.0, f, 0.0)
    a_fl[...] += jnp.sum(f).reshape(1, 1)

    @pl.when((b == nB - 1) & (pt == nPT - 1))
    def _():
        out_ref[...] = (a_fl[...] / jnp.maximum(sm_ref[...], 1.0)
                        + sd_ref[...] / jnp.maximum(sp_ref[...] * 4.0, 1.0))


def kernel(predicted_locs, predicted_scores, boxes, priors_cxcy, labels):
    B, P, C = predicted_scores.shape
    NOBJ = boxes.shape[1]
    PL = (P + _L - 1) // _L
    Ppad = PL * _L
    PT = Ppad // _PT

    priors_T = priors_cxcy.T
    boxes_p = jnp.pad(boxes, ((0, 0), (0, _NPAD - NOBJ), (0, 0)))
    labels_c = jnp.pad(labels.astype(jnp.float32),
                       ((0, 0), (0, _NPAD - NOBJ)))[..., None]
    locs_T = predicted_locs.transpose(0, 2, 1)

    pfe = pl.pallas_call(
        lambda *refs: _match_kernel(P, PL, NOBJ, *refs),
        grid=(B, PL),
        in_specs=[
            pl.BlockSpec((4, _L), lambda b, l: (0, l)),
            pl.BlockSpec((1, _NPAD, 4), lambda b, l: (b, 0, 0)),
        ],
        out_specs=pl.BlockSpec((1, _NPAD, 1), lambda b, l: (b, 0, 0)),
        out_shape=jax.ShapeDtypeStruct((B, _NPAD, 1), jnp.int32),
        scratch_shapes=[
            pltpu.VMEM((_NPAD, 1), jnp.float32),
            pltpu.VMEM((_NPAD, 1), jnp.int32),
        ],
    )(priors_T, boxes_p)

    tcls, coef, sm, sp, sd = pl.pallas_call(
        lambda *refs: _prep_kernel(P, PL, B, NOBJ, *refs),
        grid=(B, PL),
        in_specs=[
            pl.BlockSpec((4, _L), lambda b, l: (0, l)),
            pl.BlockSpec((1, _NPAD, 4), lambda b, l: (b, 0, 0)),
            pl.BlockSpec((1, _NPAD, 1), lambda b, l: (b, 0, 0)),
            pl.BlockSpec((1, _NPAD, 1), lambda b, l: (b, 0, 0)),
            pl.BlockSpec((1, 4, _L), lambda b, l: (b, 0, l)),
        ],
        out_specs=[
            pl.BlockSpec((1, 1, _L), lambda b, l: (b, 0, l)),
            pl.BlockSpec((1, 1, _L), lambda b, l: (b, 0, l)),
            pl.BlockSpec((1, 1), lambda b, l: (0, 0)),
            pl.BlockSpec((1, 1), lambda b, l: (0, 0)),
            pl.BlockSpec((1, 1), lambda b, l: (0, 0)),
        ],
        out_shape=[
            jax.ShapeDtypeStruct((B, 1, Ppad), jnp.int32),
            jax.ShapeDtypeStruct((B, 1, Ppad), jnp.float32),
            jax.ShapeDtypeStruct((1, 1), jnp.float32),
            jax.ShapeDtypeStruct((1, 1), jnp.float32),
            jax.ShapeDtypeStruct((1, 1), jnp.float32),
        ],
        scratch_shapes=[
            pltpu.VMEM((1, 1), jnp.float32),
            pltpu.VMEM((1, 1), jnp.float32),
            pltpu.VMEM((1, 1), jnp.float32),
        ],
    )(priors_T, boxes_p, labels_c, pfe, locs_T)

    tcls_c = tcls.reshape(B, Ppad, 1)
    coef_c = coef.reshape(B, Ppad, 1)

    loss = pl.pallas_call(
        lambda *refs: _loss_kernel(PT, B, *refs),
        grid=(B, PT),
        in_specs=[
            pl.BlockSpec((1, _PT, C), lambda b, pt: (b, pt, 0)),
            pl.BlockSpec((1, _PT, 1), lambda b, pt: (b, pt, 0)),
            pl.BlockSpec((1, _PT, 1), lambda b, pt: (b, pt, 0)),
            pl.BlockSpec((1, 1), lambda b, pt: (0, 0)),
            pl.BlockSpec((1, 1), lambda b, pt: (0, 0)),
            pl.BlockSpec((1, 1), lambda b, pt: (0, 0)),
        ],
        out_specs=pl.BlockSpec((1, 1), lambda b, pt: (0, 0)),
        out_shape=jax.ShapeDtypeStruct((1, 1), jnp.float32),
        scratch_shapes=[pltpu.VMEM((1, 1), jnp.float32)],
    )(predicted_scores, tcls_c, coef_c, sm, sp, sd)

    return loss[0, 0]


# speedup vs baseline: 2.6925x; 1.1524x over previous
import jax
import jax.numpy as jnp
from jax import lax
from jax.experimental import pallas as pl
from jax.experimental.pallas import tpu as pltpu

_THRESH = 0.5
_ALPHA = 0.25
_L = 4096
_PT = 4096
_NPAD = 32


def _iou_lanes(priors_ref, boxes_ref):
    pr = priors_ref[...]
    pcx = pr[0:1, :]
    pcy = pr[1:2, :]
    pw = pr[2:3, :]
    ph = pr[3:4, :]
    px0 = pcx - pw * 0.5
    py0 = pcy - ph * 0.5
    px1 = pcx + pw * 0.5
    py1 = pcy + ph * 0.5
    bo = boxes_ref[0]
    bx0 = bo[:, 0:1]
    by0 = bo[:, 1:2]
    bx1 = bo[:, 2:3]
    by1 = bo[:, 3:4]
    ix0 = jnp.maximum(px0, bx0)
    iy0 = jnp.maximum(py0, by0)
    ix1 = jnp.minimum(px1, bx1)
    iy1 = jnp.minimum(py1, by1)
    inter = jnp.maximum(ix1 - ix0, 0.0) * jnp.maximum(iy1 - iy0, 0.0)
    pa = (px1 - px0) * (py1 - py0)
    ba = (bx1 - bx0) * (by1 - by0)
    ov = inter / (pa + ba - inter)
    return ov, pcx, pcy, pw, ph, bx0, by0, bx1, by1


def _match_kernel(nP, nPL, nobj, priors_ref, boxes_ref, out_ref, vacc, iacc):
    l = pl.program_id(1)
    ov = _iou_lanes(priors_ref, boxes_ref)[0]
    glob = lax.broadcasted_iota(jnp.int32, ov.shape, 1) + l * _L
    ovm = jnp.where(glob < nP, ov, -1.0)
    rmax = jnp.max(ovm, axis=1, keepdims=True)
    ridx = jnp.min(jnp.where(ovm == rmax, glob, jnp.int32(2 ** 30)),
                   axis=1, keepdims=True)

    @pl.when(l == 0)
    def _():
        vacc[...] = rmax
        iacc[...] = ridx

    @pl.when(l > 0)
    def _():
        better = rmax > vacc[...]
        iacc[...] = jnp.where(better, ridx, iacc[...])
        vacc[...] = jnp.where(better, rmax, vacc[...])

    @pl.when(l == nPL - 1)
    def _():
        soi = lax.broadcasted_iota(jnp.int32, (_NPAD, 1), 0)
        out_ref[0] = jnp.where(soi < nobj, iacc[...], -1)


def _prep_kernel(nP, nPL, nB, nobj, priors_ref, boxes_ref, labels_ref,
                 pfe_ref, locsT_ref, tcls_ref, coef_ref,
                 sm_ref, sp_ref, sd_ref, a_m, a_p, a_d):
    b = pl.program_id(0)
    l = pl.program_id(1)

    @pl.when((b == 0) & (l == 0))
    def _():
        z = jnp.zeros((1, 1), jnp.float32)
        a_m[...] = z
        a_p[...] = z
        a_d[...] = z

    ov, pcx, pcy, pw, ph, bx0, by0, bx1, by1 = _iou_lanes(priors_ref, boxes_ref)
    soi = lax.broadcasted_iota(jnp.int32, ov.shape, 0)
    ovm = jnp.where(soi < nobj, ov, -1e30)
    ovmax = jnp.max(ovm, axis=0, keepdims=True)
    obj = jnp.min(jnp.where(ovm == ovmax, soi, jnp.int32(64)),
                  axis=0, keepdims=True)

    lidx = lax.broadcasted_iota(jnp.int32, (1, _L), 1) + l * _L
    validp = lidx < nP

    matchm = pfe_ref[0] == lidx
    mo = jnp.max(jnp.where(matchm, soi, -1), axis=0, keepdims=True)
    hit = mo >= 0
    obj = jnp.where(hit, mo, obj)
    ovmax = jnp.where(hit, 1.0, ovmax)

    onehot = soi == obj
    lab = jnp.sum(jnp.where(onehot, labels_ref[0], 0.0), axis=0, keepdims=True)
    gx0 = jnp.sum(jnp.where(onehot, bx0, 0.0), axis=0, keepdims=True)
    gy0 = jnp.sum(jnp.where(onehot, by0, 0.0), axis=0, keepdims=True)
    gx1 = jnp.sum(jnp.where(onehot, bx1, 0.0), axis=0, keepdims=True)
    gy1 = jnp.sum(jnp.where(onehot, by1, 0.0), axis=0, keepdims=True)

    pos = (ovmax >= _THRESH) & validp
    neg = (ovmax < _THRESH - 0.1) & validp
    msk = pos | neg
    tcls_ref[0] = jnp.where(pos, lab, 0.0).astype(jnp.int32)
    coef_ref[0] = jnp.where(msk, jnp.where(pos, -_ALPHA, _ALPHA - 1.0), 0.0)

    bcx = (gx0 + gx1) * 0.5
    bcy = (gy0 + gy1) * 0.5
    bw = gx1 - gx0
    bh = gy1 - gy0
    t0 = (bcx - pcx) / (pw * 0.1)
    t1 = (bcy - pcy) / (ph * 0.1)
    t2 = jnp.log(bw / pw) * 5.0
    t3 = jnp.log(bh / ph) * 5.0
    lt = locsT_ref[0]
    d = (jnp.abs(lt[0:1, :] - t0) + jnp.abs(lt[1:2, :] - t1)
         + jnp.abs(lt[2:3, :] - t2) + jnp.abs(lt[3:4, :] - t3))

    a_m[...] += jnp.sum(jnp.where(msk, 1.0, 0.0)).reshape(1, 1)
    a_p[...] += jnp.sum(jnp.where(pos, 1.0, 0.0)).reshape(1, 1)
    a_d[...] += jnp.sum(jnp.where(pos, d, 0.0)).reshape(1, 1)

    @pl.when((b == nB - 1) & (l == nPL - 1))
    def _():
        sm_ref[...] = a_m[...]
        sp_ref[...] = a_p[...]
        sd_ref[...] = a_d[...]


def _loss_kernel(nPT, nB, scores_ref, tcls_ref, coef_ref,
                 sm_ref, sp_ref, sd_ref, out_ref, a_fl):
    b = pl.program_id(0)
    pt = pl.program_id(1)

    @pl.when((b == 0) & (pt == 0))
    def _():
        a_fl[...] = jnp.zeros((1, 1), jnp.float32)

    x = scores_ref[0]
    tci = tcls_ref[0]
    coef = coef_ref[0]
    mx = jnp.max(x, axis=1, keepdims=True)
    s = x - mx
    lse = jnp.log(jnp.sum(jnp.exp(s), axis=1, keepdims=True))
    cl = lax.broadcasted_iota(jnp.int32, x.shape, 1)
    st = jnp.sum(jnp.where(cl == tci, s, 0.0), axis=1, keepdims=True)
    logpt = st - lse
    om = 1.0 - jnp.exp(logpt)
    f = coef * (om * om) * logpt
    f = jnp.where(coef < 0.0, f, 0.0)
    a_fl[...] += jnp.sum(f).reshape(1, 1)

    @pl.when((b == nB - 1) & (pt == nPT - 1))
    def _():
        out_ref[...] = (a_fl[...] / jnp.maximum(sm_ref[...], 1.0)
                        + sd_ref[...] / jnp.maximum(sp_ref[...] * 4.0, 1.0))


def kernel(predicted_locs, predicted_scores, boxes, priors_cxcy, labels):
    B, P, C = predicted_scores.shape
    NOBJ = boxes.shape[1]
    PL = (P + _L - 1) // _L
    Ppad = PL * _L
    PT = Ppad // _PT

    priors_T = priors_cxcy.T
    boxes_p = jnp.pad(boxes, ((0, 0), (0, _NPAD - NOBJ), (0, 0)))
    labels_c = jnp.pad(labels.astype(jnp.float32),
                       ((0, 0), (0, _NPAD - NOBJ)))[..., None]
    locs_T = predicted_locs.transpose(0, 2, 1)

    pfe = pl.pallas_call(
        lambda *refs: _match_kernel(P, PL, NOBJ, *refs),
        grid=(B, PL),
        in_specs=[
            pl.BlockSpec((4, _L), lambda b, l: (0, l)),
            pl.BlockSpec((1, _NPAD, 4), lambda b, l: (b, 0, 0)),
        ],
        out_specs=pl.BlockSpec((1, _NPAD, 1), lambda b, l: (b, 0, 0)),
        out_shape=jax.ShapeDtypeStruct((B, _NPAD, 1), jnp.int32),
        scratch_shapes=[
            pltpu.VMEM((_NPAD, 1), jnp.float32),
            pltpu.VMEM((_NPAD, 1), jnp.int32),
        ],
    )(priors_T, boxes_p)

    tcls, coef, sm, sp, sd = pl.pallas_call(
        lambda *refs: _prep_kernel(P, PL, B, NOBJ, *refs),
        grid=(B, PL),
        in_specs=[
            pl.BlockSpec((4, _L), lambda b, l: (0, l)),
            pl.BlockSpec((1, _NPAD, 4), lambda b, l: (b, 0, 0)),
            pl.BlockSpec((1, _NPAD, 1), lambda b, l: (b, 0, 0)),
            pl.BlockSpec((1, _NPAD, 1), lambda b, l: (b, 0, 0)),
            pl.BlockSpec((1, 4, _L), lambda b, l: (b, 0, l)),
        ],
        out_specs=[
            pl.BlockSpec((1, 1, _L), lambda b, l: (b, 0, l)),
            pl.BlockSpec((1, 1, _L), lambda b, l: (b, 0, l)),
            pl.BlockSpec((1, 1), lambda b, l: (0, 0)),
            pl.BlockSpec((1, 1), lambda b, l: (0, 0)),
            pl.BlockSpec((1, 1), lambda b, l: (0, 0)),
        ],
        out_shape=[
            jax.ShapeDtypeStruct((B, 1, Ppad), jnp.int32),
            jax.ShapeDtypeStruct((B, 1, Ppad), jnp.float32),
            jax.ShapeDtypeStruct((1, 1), jnp.float32),
            jax.ShapeDtypeStruct((1, 1), jnp.float32),
            jax.ShapeDtypeStruct((1, 1), jnp.float32),
        ],
        scratch_shapes=[
            pltpu.VMEM((1, 1), jnp.float32),
            pltpu.VMEM((1, 1), jnp.float32),
            pltpu.VMEM((1, 1), jnp.float32),
        ],
    )(priors_T, boxes_p, labels_c, pfe, locs_T)

    tcls_c = tcls.reshape(B, Ppad, 1)
    coef_c = coef.reshape(B, Ppad, 1)

    loss = pl.pallas_call(
        lambda *refs: _loss_kernel(PT, B, *refs),
        grid=(B, PT),
        in_specs=[
            pl.BlockSpec((1, _PT, C), lambda b, pt: (b, pt, 0)),
            pl.BlockSpec((1, _PT, 1), lambda b, pt: (b, pt, 0)),
            pl.BlockSpec((1, _PT, 1), lambda b, pt: (b, pt, 0)),
            pl.BlockSpec((1, 1), lambda b, pt: (0, 0)),
            pl.BlockSpec((1, 1), lambda b, pt: (0, 0)),
            pl.BlockSpec((1, 1), lambda b, pt: (0, 0)),
        ],
        out_specs=pl.BlockSpec((1, 1), lambda b, pt: (0, 0)),
        out_shape=jax.ShapeDtypeStruct((1, 1), jnp.float32),
        scratch_shapes=[pltpu.VMEM((1, 1), jnp.float32)],
    )(predicted_scores, tcls_c, coef_c, sm, sp, sd)

    return loss[0, 0]


# speedup vs baseline: 2.7951x; 1.0381x over previous
import jax
import jax.numpy as jnp
from jax import lax
from jax.experimental import pallas as pl
from jax.experimental.pallas import tpu as pltpu

_THRESH = 0.5
_ALPHA = 0.25
_L = 8192
_PT = 8192
_NPAD = 32


def _iou_lanes(priors_ref, boxes_ref):
    pr = priors_ref[...]
    pcx = pr[0:1, :]
    pcy = pr[1:2, :]
    pw = pr[2:3, :]
    ph = pr[3:4, :]
    px0 = pcx - pw * 0.5
    py0 = pcy - ph * 0.5
    px1 = pcx + pw * 0.5
    py1 = pcy + ph * 0.5
    bo = boxes_ref[0]
    bx0 = bo[:, 0:1]
    by0 = bo[:, 1:2]
    bx1 = bo[:, 2:3]
    by1 = bo[:, 3:4]
    ix0 = jnp.maximum(px0, bx0)
    iy0 = jnp.maximum(py0, by0)
    ix1 = jnp.minimum(px1, bx1)
    iy1 = jnp.minimum(py1, by1)
    inter = jnp.maximum(ix1 - ix0, 0.0) * jnp.maximum(iy1 - iy0, 0.0)
    pa = (px1 - px0) * (py1 - py0)
    ba = (bx1 - bx0) * (by1 - by0)
    ov = inter / (pa + ba - inter)
    return ov, pcx, pcy, pw, ph, bx0, by0, bx1, by1


def _match_kernel(nP, nPL, nobj, priors_ref, boxes_ref, out_ref, vacc, iacc):
    l = pl.program_id(1)
    ov = _iou_lanes(priors_ref, boxes_ref)[0]
    glob = lax.broadcasted_iota(jnp.int32, ov.shape, 1) + l * _L
    ovm = jnp.where(glob < nP, ov, -1.0)
    rmax = jnp.max(ovm, axis=1, keepdims=True)
    ridx = jnp.min(jnp.where(ovm == rmax, glob, jnp.int32(2 ** 30)),
                   axis=1, keepdims=True)

    @pl.when(l == 0)
    def _():
        vacc[...] = rmax
        iacc[...] = ridx

    @pl.when(l > 0)
    def _():
        better = rmax > vacc[...]
        iacc[...] = jnp.where(better, ridx, iacc[...])
        vacc[...] = jnp.where(better, rmax, vacc[...])

    @pl.when(l == nPL - 1)
    def _():
        soi = lax.broadcasted_iota(jnp.int32, (_NPAD, 1), 0)
        out_ref[0] = jnp.where(soi < nobj, iacc[...], -1)


def _prep_kernel(nP, nPL, nB, nobj, priors_ref, boxes_ref, labels_ref,
                 pfe_ref, locsT_ref, tcls_ref, coef_ref,
                 sm_ref, sp_ref, sd_ref, a_m, a_p, a_d):
    b = pl.program_id(0)
    l = pl.program_id(1)

    @pl.when((b == 0) & (l == 0))
    def _():
        z = jnp.zeros((1, 1), jnp.float32)
        a_m[...] = z
        a_p[...] = z
        a_d[...] = z

    ov, pcx, pcy, pw, ph, bx0, by0, bx1, by1 = _iou_lanes(priors_ref, boxes_ref)
    soi = lax.broadcasted_iota(jnp.int32, ov.shape, 0)
    ovm = jnp.where(soi < nobj, ov, -1e30)
    ovmax = jnp.max(ovm, axis=0, keepdims=True)
    obj = jnp.min(jnp.where(ovm == ovmax, soi, jnp.int32(64)),
                  axis=0, keepdims=True)

    lidx = lax.broadcasted_iota(jnp.int32, (1, _L), 1) + l * _L
    validp = lidx < nP

    matchm = pfe_ref[0] == lidx
    mo = jnp.max(jnp.where(matchm, soi, -1), axis=0, keepdims=True)
    hit = mo >= 0
    obj = jnp.where(hit, mo, obj)
    ovmax = jnp.where(hit, 1.0, ovmax)

    onehot = soi == obj
    lab = jnp.sum(jnp.where(onehot, labels_ref[0], 0.0), axis=0, keepdims=True)
    gx0 = jnp.sum(jnp.where(onehot, bx0, 0.0), axis=0, keepdims=True)
    gy0 = jnp.sum(jnp.where(onehot, by0, 0.0), axis=0, keepdims=True)
    gx1 = jnp.sum(jnp.where(onehot, bx1, 0.0), axis=0, keepdims=True)
    gy1 = jnp.sum(jnp.where(onehot, by1, 0.0), axis=0, keepdims=True)

    pos = (ovmax >= _THRESH) & validp
    neg = (ovmax < _THRESH - 0.1) & validp
    msk = pos | neg
    tcls_ref[0] = jnp.where(pos, lab, 0.0).astype(jnp.int32)
    coef_ref[0] = jnp.where(msk, jnp.where(pos, -_ALPHA, _ALPHA - 1.0), 0.0)

    bcx = (gx0 + gx1) * 0.5
    bcy = (gy0 + gy1) * 0.5
    bw = gx1 - gx0
    bh = gy1 - gy0
    t0 = (bcx - pcx) / (pw * 0.1)
    t1 = (bcy - pcy) / (ph * 0.1)
    t2 = jnp.log(bw / pw) * 5.0
    t3 = jnp.log(bh / ph) * 5.0
    lt = locsT_ref[0]
    d = (jnp.abs(lt[0:1, :] - t0) + jnp.abs(lt[1:2, :] - t1)
         + jnp.abs(lt[2:3, :] - t2) + jnp.abs(lt[3:4, :] - t3))

    a_m[...] += jnp.sum(jnp.where(msk, 1.0, 0.0)).reshape(1, 1)
    a_p[...] += jnp.sum(jnp.where(pos, 1.0, 0.0)).reshape(1, 1)
    a_d[...] += jnp.sum(jnp.where(pos, d, 0.0)).reshape(1, 1)

    @pl.when((b == nB - 1) & (l == nPL - 1))
    def _():
        sm_ref[...] = a_m[...]
        sp_ref[...] = a_p[...]
        sd_ref[...] = a_d[...]


def _loss_kernel(nPT, nB, scores_ref, tcls_ref, coef_ref,
                 sm_ref, sp_ref, sd_ref, out_ref, a_fl):
    b = pl.program_id(0)
    pt = pl.program_id(1)

    @pl.when((b == 0) & (pt == 0))
    def _():
        a_fl[...] = jnp.zeros((1, 1), jnp.float32)

    x = scores_ref[0]
    tci = tcls_ref[0]
    coef = coef_ref[0]
    mx = jnp.max(x, axis=1, keepdims=True)
    s = x - mx
    lse = jnp.log(jnp.sum(jnp.exp(s), axis=1, keepdims=True))
    cl = lax.broadcasted_iota(jnp.int32, x.shape, 1)
    st = jnp.sum(jnp.where(cl == tci, s, 0.0), axis=1, keepdims=True)
    logpt = st - lse
    om = 1.0 - jnp.exp(logpt)
    f = coef * (om * om) * logpt
    f = jnp.where(coef < 0.0, f, 0.0)
    a_fl[...] += jnp.sum(f).reshape(1, 1)

    @pl.when((b == nB - 1) & (pt == nPT - 1))
    def _():
        out_ref[...] = (a_fl[...] / jnp.maximum(sm_ref[...], 1.0)
                        + sd_ref[...] / jnp.maximum(sp_ref[...] * 4.0, 1.0))


def kernel(predicted_locs, predicted_scores, boxes, priors_cxcy, labels):
    B, P, C = predicted_scores.shape
    NOBJ = boxes.shape[1]
    PL = (P + _L - 1) // _L
    Ppad = PL * _L
    PT = Ppad // _PT

    priors_T = priors_cxcy.T
    boxes_p = jnp.pad(boxes, ((0, 0), (0, _NPAD - NOBJ), (0, 0)))
    labels_c = jnp.pad(labels.astype(jnp.float32),
                       ((0, 0), (0, _NPAD - NOBJ)))[..., None]
    locs_T = predicted_locs.transpose(0, 2, 1)

    pfe = pl.pallas_call(
        lambda *refs: _match_kernel(P, PL, NOBJ, *refs),
        grid=(B, PL),
        in_specs=[
            pl.BlockSpec((4, _L), lambda b, l: (0, l)),
            pl.BlockSpec((1, _NPAD, 4), lambda b, l: (b, 0, 0)),
        ],
        out_specs=pl.BlockSpec((1, _NPAD, 1), lambda b, l: (b, 0, 0)),
        out_shape=jax.ShapeDtypeStruct((B, _NPAD, 1), jnp.int32),
        scratch_shapes=[
            pltpu.VMEM((_NPAD, 1), jnp.float32),
            pltpu.VMEM((_NPAD, 1), jnp.int32),
        ],
    )(priors_T, boxes_p)

    tcls, coef, sm, sp, sd = pl.pallas_call(
        lambda *refs: _prep_kernel(P, PL, B, NOBJ, *refs),
        grid=(B, PL),
        in_specs=[
            pl.BlockSpec((4, _L), lambda b, l: (0, l)),
            pl.BlockSpec((1, _NPAD, 4), lambda b, l: (b, 0, 0)),
            pl.BlockSpec((1, _NPAD, 1), lambda b, l: (b, 0, 0)),
            pl.BlockSpec((1, _NPAD, 1), lambda b, l: (b, 0, 0)),
            pl.BlockSpec((1, 4, _L), lambda b, l: (b, 0, l)),
        ],
        out_specs=[
            pl.BlockSpec((1, 1, _L), lambda b, l: (b, 0, l)),
            pl.BlockSpec((1, 1, _L), lambda b, l: (b, 0, l)),
            pl.BlockSpec((1, 1), lambda b, l: (0, 0)),
            pl.BlockSpec((1, 1), lambda b, l: (0, 0)),
            pl.BlockSpec((1, 1), lambda b, l: (0, 0)),
        ],
        out_shape=[
            jax.ShapeDtypeStruct((B, 1, Ppad), jnp.int32),
            jax.ShapeDtypeStruct((B, 1, Ppad), jnp.float32),
            jax.ShapeDtypeStruct((1, 1), jnp.float32),
            jax.ShapeDtypeStruct((1, 1), jnp.float32),
            jax.ShapeDtypeStruct((1, 1), jnp.float32),
        ],
        scratch_shapes=[
            pltpu.VMEM((1, 1), jnp.float32),
            pltpu.VMEM((1, 1), jnp.float32),
            pltpu.VMEM((1, 1), jnp.float32),
        ],
    )(priors_T, boxes_p, labels_c, pfe, locs_T)

    tcls_c = tcls.reshape(B, Ppad, 1)
    coef_c = coef.reshape(B, Ppad, 1)

    loss = pl.pallas_call(
        lambda *refs: _loss_kernel(PT, B, *refs),
        grid=(B, PT),
        in_specs=[
            pl.BlockSpec((1, _PT, C), lambda b, pt: (b, pt, 0)),
            pl.BlockSpec((1, _PT, 1), lambda b, pt: (b, pt, 0)),
            pl.BlockSpec((1, _PT, 1), lambda b, pt: (b, pt, 0)),
            pl.BlockSpec((1, 1), lambda b, pt: (0, 0)),
            pl.BlockSpec((1, 1), lambda b, pt: (0, 0)),
            pl.BlockSpec((1, 1), lambda b, pt: (0, 0)),
        ],
        out_specs=pl.BlockSpec((1, 1), lambda b, pt: (0, 0)),
        out_shape=jax.ShapeDtypeStruct((1, 1), jnp.float32),
        scratch_shapes=[pltpu.VMEM((1, 1), jnp.float32)],
    )(predicted_scores, tcls_c, coef_c, sm, sp, sd)

    return loss[0, 0]


# speedup vs baseline: 2.9960x; 1.0719x over previous
import jax
import jax.numpy as jnp
from jax import lax
from jax.experimental import pallas as pl
from jax.experimental.pallas import tpu as pltpu

_THRESH = 0.5
_ALPHA = 0.25
_L = 8192
_PT = 8192


def _iou_lanes(priors_ref, boxes_ref):
    pr = priors_ref[...]
    pcx = pr[0:1, :]
    pcy = pr[1:2, :]
    pw = pr[2:3, :]
    ph = pr[3:4, :]
    px0 = pcx - pw * 0.5
    py0 = pcy - ph * 0.5
    px1 = pcx + pw * 0.5
    py1 = pcy + ph * 0.5
    bo = boxes_ref[0]
    bx0 = bo[:, 0:1]
    by0 = bo[:, 1:2]
    bx1 = bo[:, 2:3]
    by1 = bo[:, 3:4]
    ix0 = jnp.maximum(px0, bx0)
    iy0 = jnp.maximum(py0, by0)
    ix1 = jnp.minimum(px1, bx1)
    iy1 = jnp.minimum(py1, by1)
    inter = jnp.maximum(ix1 - ix0, 0.0) * jnp.maximum(iy1 - iy0, 0.0)
    pa = (px1 - px0) * (py1 - py0)
    ba = (bx1 - bx0) * (by1 - by0)
    ov = inter / (pa + ba - inter)
    return ov, pcx, pcy, pw, ph, bx0, by0, bx1, by1


def _matchprep_kernel(nP, nPL, nB, nobj,
                      priors_ref, boxes_ref, labels_ref, locsT_ref,
                      tcls_ref, coef_ref, sm_ref, sp_ref, sd_ref,
                      vacc, iacc, ovx, objs, a_m, a_p, a_d):
    b = pl.program_id(0)
    l = pl.program_id(1)

    @pl.when((b == 0) & (l == 0))
    def _():
        z = jnp.zeros((1, 1), jnp.float32)
        a_m[...] = z
        a_p[...] = z
        a_d[...] = z

    @pl.when(l < nPL)
    def _():
        ov = _iou_lanes(priors_ref, boxes_ref)[0]
        glob = lax.broadcasted_iota(jnp.int32, ov.shape, 1) + l * _L
        ovm = jnp.where(glob < nP, ov, -1.0)
        soi = lax.broadcasted_iota(jnp.int32, ov.shape, 0)
        ovmax_t = jnp.max(ovm, axis=0, keepdims=True)
        obj_t = jnp.min(jnp.where(ovm == ovmax_t, soi, jnp.int32(64)),
                        axis=0, keepdims=True)
        ovx[:, pl.ds(l * _L, _L)] = ovmax_t
        objs[:, pl.ds(l * _L, _L)] = obj_t
        rmax = jnp.max(ovm, axis=1, keepdims=True)
        ridx = jnp.min(jnp.where(ovm == rmax, glob, jnp.int32(2 ** 30)),
                       axis=1, keepdims=True)

        @pl.when(l == 0)
        def _():
            vacc[...] = rmax
            iacc[...] = ridx

        @pl.when(l > 0)
        def _():
            better = rmax > vacc[...]
            iacc[...] = jnp.where(better, ridx, iacc[...])
            vacc[...] = jnp.where(better, rmax, vacc[...])

    @pl.when(l >= nPL)
    def _():
        t2 = l - nPL
        _, pcx, pcy, pw, ph, bx0, by0, bx1, by1 = _iou_lanes(priors_ref,
                                                             boxes_ref)
        ovmax = ovx[:, pl.ds(t2 * _L, _L)]
        obj = objs[:, pl.ds(t2 * _L, _L)]
        lidx = lax.broadcasted_iota(jnp.int32, (1, _L), 1) + t2 * _L
        validp = lidx < nP

        matchm = iacc[...] == lidx
        soi = lax.broadcasted_iota(jnp.int32, matchm.shape, 0)
        mo = jnp.max(jnp.where(matchm, soi, -1), axis=0, keepdims=True)
        hit = mo >= 0
        obj = jnp.where(hit, mo, obj)
        ovmax = jnp.where(hit, 1.0, ovmax)

        onehot = soi == obj
        labf = labels_ref[0].astype(jnp.float32)
        lab = jnp.sum(jnp.where(onehot, labf, 0.0), axis=0, keepdims=True)
        gx0 = jnp.sum(jnp.where(onehot, bx0, 0.0), axis=0, keepdims=True)
        gy0 = jnp.sum(jnp.where(onehot, by0, 0.0), axis=0, keepdims=True)
        gx1 = jnp.sum(jnp.where(onehot, bx1, 0.0), axis=0, keepdims=True)
        gy1 = jnp.sum(jnp.where(onehot, by1, 0.0), axis=0, keepdims=True)

        pos = (ovmax >= _THRESH) & validp
        neg = (ovmax < _THRESH - 0.1) & validp
        msk = pos | neg
        tcls_ref[0] = jnp.where(pos, lab, 0.0).astype(jnp.int32)
        coef_ref[0] = jnp.where(msk, jnp.where(pos, -_ALPHA, _ALPHA - 1.0),
                                0.0)

        bcx = (gx0 + gx1) * 0.5
        bcy = (gy0 + gy1) * 0.5
        bw = gx1 - gx0
        bh = gy1 - gy0
        t0 = (bcx - pcx) / (pw * 0.1)
        t1 = (bcy - pcy) / (ph * 0.1)
        t2_ = jnp.log(bw / pw) * 5.0
        t3 = jnp.log(bh / ph) * 5.0
        lt = locsT_ref[0]
        d = (jnp.abs(lt[0:1, :] - t0) + jnp.abs(lt[1:2, :] - t1)
             + jnp.abs(lt[2:3, :] - t2_) + jnp.abs(lt[3:4, :] - t3))

        a_m[...] += jnp.sum(jnp.where(msk, 1.0, 0.0)).reshape(1, 1)
        a_p[...] += jnp.sum(jnp.where(pos, 1.0, 0.0)).reshape(1, 1)
        a_d[...] += jnp.sum(jnp.where(pos, d, 0.0)).reshape(1, 1)

    @pl.when((b == nB - 1) & (l == 2 * nPL - 1))
    def _():
        sm_ref[...] = a_m[...]
        sp_ref[...] = a_p[...]
        sd_ref[...] = a_d[...]


def _loss_kernel(nPT, nB, scores_ref, tcls_ref, coef_ref,
                 sm_ref, sp_ref, sd_ref, out_ref, a_fl):
    b = pl.program_id(0)
    pt = pl.program_id(1)

    @pl.when((b == 0) & (pt == 0))
    def _():
        a_fl[...] = jnp.zeros((1, 1), jnp.float32)

    x = scores_ref[0]
    tci = tcls_ref[0]
    coef = coef_ref[0]
    mx = jnp.max(x, axis=1, keepdims=True)
    s = x - mx
    lse = jnp.log(jnp.sum(jnp.exp(s), axis=1, keepdims=True))
    cl = lax.broadcasted_iota(jnp.int32, x.shape, 1)
    st = jnp.sum(jnp.where(cl == tci, s, 0.0), axis=1, keepdims=True)
    logpt = st - lse
    om = 1.0 - jnp.exp(logpt)
    f = coef * (om * om) * logpt
    f = jnp.where(coef < 0.0, f, 0.0)
    a_fl[...] += jnp.sum(f).reshape(1, 1)

    @pl.when((b == nB - 1) & (pt == nPT - 1))
    def _():
        out_ref[...] = (a_fl[...] / jnp.maximum(sm_ref[...], 1.0)
                        + sd_ref[...] / jnp.maximum(sp_ref[...] * 4.0, 1.0))


def kernel(predicted_locs, predicted_scores, boxes, priors_cxcy, labels):
    B, P, C = predicted_scores.shape
    NOBJ = boxes.shape[1]
    PL = (P + _L - 1) // _L
    Ppad = PL * _L
    PT = Ppad // _PT

    priors_T = priors_cxcy.T
    labels_c = labels[..., None]
    locs_T = predicted_locs.transpose(0, 2, 1)

    tcls, coef, sm, sp, sd = pl.pallas_call(
        lambda *refs: _matchprep_kernel(P, PL, B, NOBJ, *refs),
        grid=(B, 2 * PL),
        in_specs=[
            pl.BlockSpec((4, _L), lambda b, l: (0, lax.rem(l, PL))),
            pl.BlockSpec((1, NOBJ, 4), lambda b, l: (b, 0, 0)),
            pl.BlockSpec((1, NOBJ, 1), lambda b, l: (b, 0, 0)),
            pl.BlockSpec((1, 4, _L),
                         lambda b, l: (b, 0, jnp.maximum(l - PL, 0))),
        ],
        out_specs=[
            pl.BlockSpec((1, 1, _L),
                         lambda b, l: (b, 0, jnp.maximum(l - PL, 0))),
            pl.BlockSpec((1, 1, _L),
                         lambda b, l: (b, 0, jnp.maximum(l - PL, 0))),
            pl.BlockSpec((1, 1), lambda b, l: (0, 0)),
            pl.BlockSpec((1, 1), lambda b, l: (0, 0)),
            pl.BlockSpec((1, 1), lambda b, l: (0, 0)),
        ],
        out_shape=[
            jax.ShapeDtypeStruct((B, 1, Ppad), jnp.int32),
            jax.ShapeDtypeStruct((B, 1, Ppad), jnp.float32),
            jax.ShapeDtypeStruct((1, 1), jnp.float32),
            jax.ShapeDtypeStruct((1, 1), jnp.float32),
            jax.ShapeDtypeStruct((1, 1), jnp.float32),
        ],
        scratch_shapes=[
            pltpu.VMEM((NOBJ, 1), jnp.float32),
            pltpu.VMEM((NOBJ, 1), jnp.int32),
            pltpu.VMEM((1, Ppad), jnp.float32),
            pltpu.VMEM((1, Ppad), jnp.int32),
            pltpu.VMEM((1, 1), jnp.float32),
            pltpu.VMEM((1, 1), jnp.float32),
            pltpu.VMEM((1, 1), jnp.float32),
        ],
    )(priors_T, boxes, labels_c, locs_T)

    tcls_c = tcls.reshape(B, Ppad, 1)
    coef_c = coef.reshape(B, Ppad, 1)

    loss = pl.pallas_call(
        lambda *refs: _loss_kernel(PT, B, *refs),
        grid=(B, PT),
        in_specs=[
            pl.BlockSpec((1, _PT, C), lambda b, pt: (b, pt, 0)),
            pl.BlockSpec((1, _PT, 1), lambda b, pt: (b, pt, 0)),
            pl.BlockSpec((1, _PT, 1), lambda b, pt: (b, pt, 0)),
            pl.BlockSpec((1, 1), lambda b, pt: (0, 0)),
            pl.BlockSpec((1, 1), lambda b, pt: (0, 0)),
            pl.BlockSpec((1, 1), lambda b, pt: (0, 0)),
        ],
        out_specs=pl.BlockSpec((1, 1), lambda b, pt: (0, 0)),
        out_shape=jax.ShapeDtypeStruct((1, 1), jnp.float32),
        scratch_shapes=[pltpu.VMEM((1, 1), jnp.float32)],
    )(predicted_scores, tcls_c, coef_c, sm, sp, sd)

    return loss[0, 0]


# speedup vs baseline: 3.3028x; 1.1024x over previous
import jax
import jax.numpy as jnp
from jax import lax
from jax.experimental import pallas as pl
from jax.experimental.pallas import tpu as pltpu

_THRESH = 0.5
_ALPHA = 0.25
_L = 8192
_PT = 8192


def _iou_lanes(priors_ref, boxes_ref):
    pr = priors_ref[...]
    pcx = pr[0:1, :]
    pcy = pr[1:2, :]
    pw = pr[2:3, :]
    ph = pr[3:4, :]
    px0 = pcx - pw * 0.5
    py0 = pcy - ph * 0.5
    px1 = pcx + pw * 0.5
    py1 = pcy + ph * 0.5
    bo = boxes_ref[0]
    bx0 = bo[:, 0:1]
    by0 = bo[:, 1:2]
    bx1 = bo[:, 2:3]
    by1 = bo[:, 3:4]
    ix0 = jnp.maximum(px0, bx0)
    iy0 = jnp.maximum(py0, by0)
    ix1 = jnp.minimum(px1, bx1)
    iy1 = jnp.minimum(py1, by1)
    inter = jnp.maximum(ix1 - ix0, 0.0) * jnp.maximum(iy1 - iy0, 0.0)
    pa = (px1 - px0) * (py1 - py0)
    ba = (bx1 - bx0) * (by1 - by0)
    ov = inter / (pa + ba - inter)
    return ov, pcx, pcy, pw, ph, bx0, by0, bx1, by1


def _matchprep_kernel(nP, nPL, nB, nobj,
                      priors_ref, boxes_ref, labels_ref, locsT_ref,
                      code_ref, sm_ref, sp_ref, sd_ref,
                      vacc, iacc, ovx, objs, a_m, a_p, a_d):
    b = pl.program_id(0)
    l = pl.program_id(1)

    @pl.when((b == 0) & (l == 0))
    def _():
        z = jnp.zeros((1, 1), jnp.float32)
        a_m[...] = z
        a_p[...] = z
        a_d[...] = z

    @pl.when(l < nPL)
    def _():
        ov = _iou_lanes(priors_ref, boxes_ref)[0]
        glob = lax.broadcasted_iota(jnp.int32, ov.shape, 1) + l * _L
        ovm = jnp.where(glob < nP, ov, -1.0)
        soi = lax.broadcasted_iota(jnp.int32, ov.shape, 0)
        ovmax_t = jnp.max(ovm, axis=0, keepdims=True)
        obj_t = jnp.min(jnp.where(ovm == ovmax_t, soi, jnp.int32(64)),
                        axis=0, keepdims=True)
        ovx[:, pl.ds(l * _L, _L)] = ovmax_t
        objs[:, pl.ds(l * _L, _L)] = obj_t
        rmax = jnp.max(ovm, axis=1, keepdims=True)
        ridx = jnp.min(jnp.where(ovm == rmax, glob, jnp.int32(2 ** 30)),
                       axis=1, keepdims=True)

        @pl.when(l == 0)
        def _():
            vacc[...] = rmax
            iacc[...] = ridx

        @pl.when(l > 0)
        def _():
            better = rmax > vacc[...]
            iacc[...] = jnp.where(better, ridx, iacc[...])
            vacc[...] = jnp.where(better, rmax, vacc[...])

    @pl.when(l >= nPL)
    def _():
        t2 = l - nPL
        _, pcx, pcy, pw, ph, bx0, by0, bx1, by1 = _iou_lanes(priors_ref,
                                                             boxes_ref)
        ovmax = ovx[:, pl.ds(t2 * _L, _L)]
        obj = objs[:, pl.ds(t2 * _L, _L)]
        lidx = lax.broadcasted_iota(jnp.int32, (1, _L), 1) + t2 * _L
        validp = lidx < nP

        matchm = iacc[...] == lidx
        soi = lax.broadcasted_iota(jnp.int32, matchm.shape, 0)
        mo = jnp.max(jnp.where(matchm, soi, -1), axis=0, keepdims=True)
        hit = mo >= 0
        obj = jnp.where(hit, mo, obj)
        ovmax = jnp.where(hit, 1.0, ovmax)

        onehot = soi == obj
        labf = labels_ref[0].astype(jnp.float32)
        lab = jnp.sum(jnp.where(onehot, labf, 0.0), axis=0, keepdims=True)
        gx0 = jnp.sum(jnp.where(onehot, bx0, 0.0), axis=0, keepdims=True)
        gy0 = jnp.sum(jnp.where(onehot, by0, 0.0), axis=0, keepdims=True)
        gx1 = jnp.sum(jnp.where(onehot, bx1, 0.0), axis=0, keepdims=True)
        gy1 = jnp.sum(jnp.where(onehot, by1, 0.0), axis=0, keepdims=True)

        pos = (ovmax >= _THRESH) & validp
        neg = (ovmax < _THRESH - 0.1) & validp
        msk = pos | neg
        code_ref[0] = jnp.where(pos, -(lab + _ALPHA),
                                jnp.where(neg, _ALPHA - 1.0, 0.0))

        bcx = (gx0 + gx1) * 0.5
        bcy = (gy0 + gy1) * 0.5
        bw = gx1 - gx0
        bh = gy1 - gy0
        t0 = (bcx - pcx) / (pw * 0.1)
        t1 = (bcy - pcy) / (ph * 0.1)
        t2_ = jnp.log(bw / pw) * 5.0
        t3 = jnp.log(bh / ph) * 5.0
        lt = locsT_ref[0]
        d = (jnp.abs(lt[0:1, :] - t0) + jnp.abs(lt[1:2, :] - t1)
             + jnp.abs(lt[2:3, :] - t2_) + jnp.abs(lt[3:4, :] - t3))

        a_m[...] += jnp.sum(jnp.where(msk, 1.0, 0.0)).reshape(1, 1)
        a_p[...] += jnp.sum(jnp.where(pos, 1.0, 0.0)).reshape(1, 1)
        a_d[...] += jnp.sum(jnp.where(pos, d, 0.0)).reshape(1, 1)

    @pl.when((b == nB - 1) & (l == 2 * nPL - 1))
    def _():
        sm_ref[...] = a_m[...]
        sp_ref[...] = a_p[...]
        sd_ref[...] = a_d[...]


def _loss_kernel(nPT, nB, scores_ref, code_ref,
                 sm_ref, sp_ref, sd_ref, out_ref, a_fl):
    b = pl.program_id(0)
    pt = pl.program_id(1)

    @pl.when((b == 0) & (pt == 0))
    def _():
        a_fl[...] = jnp.zeros((1, 1), jnp.float32)

    x = scores_ref[0]
    c = code_ref[0]
    tci = jnp.floor(-c).astype(jnp.int32)
    coef = jnp.where(c < -1.0, -_ALPHA, c)
    mx = jnp.max(x, axis=1, keepdims=True)
    s = x - mx
    lse = jnp.log(jnp.sum(jnp.exp(s), axis=1, keepdims=True))
    cl = lax.broadcasted_iota(jnp.int32, x.shape, 1)
    st = jnp.sum(jnp.where(cl == tci, s, 0.0), axis=1, keepdims=True)
    logpt = st - lse
    om = 1.0 - jnp.exp(logpt)
    f = coef * (om * om) * logpt
    f = jnp.where(c < 0.0, f, 0.0)
    a_fl[...] += jnp.sum(f).reshape(1, 1)

    @pl.when((b == nB - 1) & (pt == nPT - 1))
    def _():
        out_ref[...] = (a_fl[...] / jnp.maximum(sm_ref[...], 1.0)
                        + sd_ref[...] / jnp.maximum(sp_ref[...] * 4.0, 1.0))


def kernel(predicted_locs, predicted_scores, boxes, priors_cxcy, labels):
    B, P, C = predicted_scores.shape
    NOBJ = boxes.shape[1]
    PL = (P + _L - 1) // _L
    Ppad = PL * _L
    PT = Ppad // _PT

    priors_T = priors_cxcy.T
    labels_c = labels[..., None]
    locs_T = predicted_locs.transpose(0, 2, 1)

    code, sm, sp, sd = pl.pallas_call(
        lambda *refs: _matchprep_kernel(P, PL, B, NOBJ, *refs),
        grid=(B, 2 * PL),
        in_specs=[
            pl.BlockSpec((4, _L), lambda b, l: (0, lax.rem(l, PL))),
            pl.BlockSpec((1, NOBJ, 4), lambda b, l: (b, 0, 0)),
            pl.BlockSpec((1, NOBJ, 1), lambda b, l: (b, 0, 0)),
            pl.BlockSpec((1, 4, _L),
                         lambda b, l: (b, 0, jnp.maximum(l - PL, 0))),
        ],
        out_specs=[
            pl.BlockSpec((1, 1, _L),
                         lambda b, l: (b, 0, jnp.maximum(l - PL, 0))),
            pl.BlockSpec((1, 1), lambda b, l: (0, 0)),
            pl.BlockSpec((1, 1), lambda b, l: (0, 0)),
            pl.BlockSpec((1, 1), lambda b, l: (0, 0)),
        ],
        out_shape=[
            jax.ShapeDtypeStruct((B, 1, Ppad), jnp.float32),
            jax.ShapeDtypeStruct((1, 1), jnp.float32),
            jax.ShapeDtypeStruct((1, 1), jnp.float32),
            jax.ShapeDtypeStruct((1, 1), jnp.float32),
        ],
        scratch_shapes=[
            pltpu.VMEM((NOBJ, 1), jnp.float32),
            pltpu.VMEM((NOBJ, 1), jnp.int32),
            pltpu.VMEM((1, Ppad), jnp.float32),
            pltpu.VMEM((1, Ppad), jnp.int32),
            pltpu.VMEM((1, 1), jnp.float32),
            pltpu.VMEM((1, 1), jnp.float32),
            pltpu.VMEM((1, 1), jnp.float32),
        ],
    )(priors_T, boxes, labels_c, locs_T)

    code_c = code.reshape(B, Ppad, 1)

    loss = pl.pallas_call(
        lambda *refs: _loss_kernel(PT, B, *refs),
        grid=(B, PT),
        in_specs=[
            pl.BlockSpec((1, _PT, C), lambda b, pt: (b, pt, 0)),
            pl.BlockSpec((1, _PT, 1), lambda b, pt: (b, pt, 0)),
            pl.BlockSpec((1, 1), lambda b, pt: (0, 0)),
            pl.BlockSpec((1, 1), lambda b, pt: (0, 0)),
            pl.BlockSpec((1, 1), lambda b, pt: (0, 0)),
        ],
        out_specs=pl.BlockSpec((1, 1), lambda b, pt: (0, 0)),
        out_shape=jax.ShapeDtypeStruct((1, 1), jnp.float32),
        scratch_shapes=[pltpu.VMEM((1, 1), jnp.float32)],
    )(predicted_scores, code_c, sm, sp, sd)

    return loss[0, 0]


# speedup vs baseline: 3.6715x; 1.1117x over previous
import jax
import jax.numpy as jnp
from jax import lax
from jax.experimental import pallas as pl
from jax.experimental.pallas import tpu as pltpu

_THRESH = 0.5
_ALPHA = 0.25
_L = 8192
_PT = 8192


def _iou_lanes(priors_ref, boxes_ref):
    pr = priors_ref[...]
    pcx = pr[0:1, :]
    pcy = pr[1:2, :]
    pw = pr[2:3, :]
    ph = pr[3:4, :]
    px0 = pcx - pw * 0.5
    py0 = pcy - ph * 0.5
    px1 = pcx + pw * 0.5
    py1 = pcy + ph * 0.5
    bo = boxes_ref[0]
    bx0 = bo[:, 0:1]
    by0 = bo[:, 1:2]
    bx1 = bo[:, 2:3]
    by1 = bo[:, 3:4]
    ix0 = jnp.maximum(px0, bx0)
    iy0 = jnp.maximum(py0, by0)
    ix1 = jnp.minimum(px1, bx1)
    iy1 = jnp.minimum(py1, by1)
    inter = jnp.maximum(ix1 - ix0, 0.0) * jnp.maximum(iy1 - iy0, 0.0)
    pa = (px1 - px0) * (py1 - py0)
    ba = (bx1 - bx0) * (by1 - by0)
    ov = inter / (pa + ba - inter)
    return ov, pcx, pcy, pw, ph, bx0, by0, bx1, by1


def _matchprep_kernel(nP, nPL, nB, nobj,
                      priors_ref, boxes_ref, labels_ref, locsT_ref,
                      code_ref, sm_ref, sp_ref, sd_ref,
                      vacc, iacc, ovx, objs, a_m, a_p, a_d):
    b = pl.program_id(0)
    l = pl.program_id(1)

    @pl.when((b == 0) & (l == 0))
    def _():
        z = jnp.zeros((1, 1), jnp.float32)
        a_m[...] = z
        a_p[...] = z
        a_d[...] = z

    @pl.when(l < nPL)
    def _():
        ov = _iou_lanes(priors_ref, boxes_ref)[0]
        glob = lax.broadcasted_iota(jnp.int32, ov.shape, 1) + l * _L
        ovm = jnp.where(glob < nP, ov, -1.0)
        soi = lax.broadcasted_iota(jnp.int32, ov.shape, 0)
        ovmax_t = jnp.max(ovm, axis=0, keepdims=True)
        obj_t = jnp.min(jnp.where(ovm == ovmax_t, soi, jnp.int32(64)),
                        axis=0, keepdims=True)
        ovx[:, pl.ds(l * _L, _L)] = ovmax_t
        objs[:, pl.ds(l * _L, _L)] = obj_t
        rmax = jnp.max(ovm, axis=1, keepdims=True)
        ridx = jnp.min(jnp.where(ovm == rmax, glob, jnp.int32(2 ** 30)),
                       axis=1, keepdims=True)

        @pl.when(l == 0)
        def _():
            vacc[...] = rmax
            iacc[...] = ridx

        @pl.when(l > 0)
        def _():
            better = rmax > vacc[...]
            iacc[...] = jnp.where(better, ridx, iacc[...])
            vacc[...] = jnp.where(better, rmax, vacc[...])

    @pl.when(l >= nPL)
    def _():
        t2 = l - nPL
        _, pcx, pcy, pw, ph, bx0, by0, bx1, by1 = _iou_lanes(priors_ref,
                                                             boxes_ref)
        ovmax = ovx[:, pl.ds(t2 * _L, _L)]
        obj = objs[:, pl.ds(t2 * _L, _L)]
        lidx = lax.broadcasted_iota(jnp.int32, (1, _L), 1) + t2 * _L
        validp = lidx < nP

        matchm = iacc[...] == lidx
        soi = lax.broadcasted_iota(jnp.int32, matchm.shape, 0)
        mo = jnp.max(jnp.where(matchm, soi, -1), axis=0, keepdims=True)
        hit = mo >= 0
        obj = jnp.where(hit, mo, obj)
        ovmax = jnp.where(hit, 1.0, ovmax)

        onehot = soi == obj
        labf = labels_ref[0].astype(jnp.float32)
        lab = jnp.sum(jnp.where(onehot, labf, 0.0), axis=0, keepdims=True)
        gx0 = jnp.sum(jnp.where(onehot, bx0, 0.0), axis=0, keepdims=True)
        gy0 = jnp.sum(jnp.where(onehot, by0, 0.0), axis=0, keepdims=True)
        gx1 = jnp.sum(jnp.where(onehot, bx1, 0.0), axis=0, keepdims=True)
        gy1 = jnp.sum(jnp.where(onehot, by1, 0.0), axis=0, keepdims=True)

        pos = (ovmax >= _THRESH) & validp
        neg = (ovmax < _THRESH - 0.1) & validp
        msk = pos | neg
        code_ref[0] = jnp.where(pos, -(lab + _ALPHA),
                                jnp.where(neg, _ALPHA - 1.0, 0.0))

        bcx = (gx0 + gx1) * 0.5
        bcy = (gy0 + gy1) * 0.5
        bw = gx1 - gx0
        bh = gy1 - gy0
        t0 = (bcx - pcx) / (pw * 0.1)
        t1 = (bcy - pcy) / (ph * 0.1)
        t2_ = jnp.log(bw / pw) * 5.0
        t3 = jnp.log(bh / ph) * 5.0
        lt = locsT_ref[0]
        d = (jnp.abs(lt[0:1, :] - t0) + jnp.abs(lt[1:2, :] - t1)
             + jnp.abs(lt[2:3, :] - t2_) + jnp.abs(lt[3:4, :] - t3))

        a_m[...] += jnp.sum(jnp.where(msk, 1.0, 0.0)).reshape(1, 1)
        a_p[...] += jnp.sum(jnp.where(pos, 1.0, 0.0)).reshape(1, 1)
        a_d[...] += jnp.sum(jnp.where(pos, d, 0.0)).reshape(1, 1)

    @pl.when((b == nB - 1) & (l == 2 * nPL - 1))
    def _():
        sm_ref[...] = a_m[...]
        sp_ref[...] = a_p[...]
        sd_ref[...] = a_d[...]


def _loss_kernel(nPT, nB, scores_ref, code_ref,
                 sm_ref, sp_ref, sd_ref, out_ref, a_fl):
    b = pl.program_id(0)
    pt = pl.program_id(1)

    @pl.when((b == 0) & (pt == 0))
    def _():
        a_fl[...] = jnp.zeros((1, 1), jnp.float32)

    x = scores_ref[0]
    c = code_ref[0].reshape(_PT, 1)
    tci = jnp.floor(-c).astype(jnp.int32)
    coef = jnp.where(c < -1.0, -_ALPHA, c)
    mx = jnp.max(x, axis=1, keepdims=True)
    s = x - mx
    lse = jnp.log(jnp.sum(jnp.exp(s), axis=1, keepdims=True))
    cl = lax.broadcasted_iota(jnp.int32, x.shape, 1)
    st = jnp.sum(jnp.where(cl == tci, s, 0.0), axis=1, keepdims=True)
    logpt = st - lse
    om = 1.0 - jnp.exp(logpt)
    f = coef * (om * om) * logpt
    f = jnp.where(c < 0.0, f, 0.0)
    a_fl[...] += jnp.sum(f).reshape(1, 1)

    @pl.when((b == nB - 1) & (pt == nPT - 1))
    def _():
        out_ref[...] = (a_fl[...] / jnp.maximum(sm_ref[...], 1.0)
                        + sd_ref[...] / jnp.maximum(sp_ref[...] * 4.0, 1.0))


def kernel(predicted_locs, predicted_scores, boxes, priors_cxcy, labels):
    B, P, C = predicted_scores.shape
    NOBJ = boxes.shape[1]
    PL = (P + _L - 1) // _L
    Ppad = PL * _L
    PT = Ppad // _PT

    priors_T = priors_cxcy.T
    labels_c = labels[..., None]
    locs_T = predicted_locs.transpose(0, 2, 1)

    code, sm, sp, sd = pl.pallas_call(
        lambda *refs: _matchprep_kernel(P, PL, B, NOBJ, *refs),
        grid=(B, 2 * PL),
        in_specs=[
            pl.BlockSpec((4, _L), lambda b, l: (0, lax.rem(l, PL))),
            pl.BlockSpec((1, NOBJ, 4), lambda b, l: (b, 0, 0)),
            pl.BlockSpec((1, NOBJ, 1), lambda b, l: (b, 0, 0)),
            pl.BlockSpec((1, 4, _L),
                         lambda b, l: (b, 0, jnp.maximum(l - PL, 0))),
        ],
        out_specs=[
            pl.BlockSpec((1, 1, _L),
                         lambda b, l: (b, 0, jnp.maximum(l - PL, 0))),
            pl.BlockSpec((1, 1), lambda b, l: (0, 0)),
            pl.BlockSpec((1, 1), lambda b, l: (0, 0)),
            pl.BlockSpec((1, 1), lambda b, l: (0, 0)),
        ],
        out_shape=[
            jax.ShapeDtypeStruct((B, 1, Ppad), jnp.float32),
            jax.ShapeDtypeStruct((1, 1), jnp.float32),
            jax.ShapeDtypeStruct((1, 1), jnp.float32),
            jax.ShapeDtypeStruct((1, 1), jnp.float32),
        ],
        scratch_shapes=[
            pltpu.VMEM((NOBJ, 1), jnp.float32),
            pltpu.VMEM((NOBJ, 1), jnp.int32),
            pltpu.VMEM((1, Ppad), jnp.float32),
            pltpu.VMEM((1, Ppad), jnp.int32),
            pltpu.VMEM((1, 1), jnp.float32),
            pltpu.VMEM((1, 1), jnp.float32),
            pltpu.VMEM((1, 1), jnp.float32),
        ],
    )(priors_T, boxes, labels_c, locs_T)

    loss = pl.pallas_call(
        lambda *refs: _loss_kernel(PT, B, *refs),
        grid=(B, PT),
        in_specs=[
            pl.BlockSpec((1, _PT, C), lambda b, pt: (b, pt, 0)),
            pl.BlockSpec((1, 1, _PT), lambda b, pt: (b, 0, pt)),
            pl.BlockSpec((1, 1), lambda b, pt: (0, 0)),
            pl.BlockSpec((1, 1), lambda b, pt: (0, 0)),
            pl.BlockSpec((1, 1), lambda b, pt: (0, 0)),
        ],
        out_specs=pl.BlockSpec((1, 1), lambda b, pt: (0, 0)),
        out_shape=jax.ShapeDtypeStruct((1, 1), jnp.float32),
        scratch_shapes=[pltpu.VMEM((1, 1), jnp.float32)],
    )(predicted_scores, code, sm, sp, sd)

    return loss[0, 0]


# speedup vs baseline: 3.6759x; 1.0012x over previous
import jax
import jax.numpy as jnp
from jax import lax
from jax.experimental import pallas as pl
from jax.experimental.pallas import tpu as pltpu

_THRESH = 0.5
_ALPHA = 0.25
_L = 8192
_PT = 8192


def _iou_lanes(priors_ref, boxes_ref):
    pr = priors_ref[...]
    pcx = pr[0:1, :]
    pcy = pr[1:2, :]
    pw = pr[2:3, :]
    ph = pr[3:4, :]
    px0 = pcx - pw * 0.5
    py0 = pcy - ph * 0.5
    px1 = pcx + pw * 0.5
    py1 = pcy + ph * 0.5
    bo = boxes_ref[0]
    bx0 = bo[:, 0:1]
    by0 = bo[:, 1:2]
    bx1 = bo[:, 2:3]
    by1 = bo[:, 3:4]
    ix0 = jnp.maximum(px0, bx0)
    iy0 = jnp.maximum(py0, by0)
    ix1 = jnp.minimum(px1, bx1)
    iy1 = jnp.minimum(py1, by1)
    inter = jnp.maximum(ix1 - ix0, 0.0) * jnp.maximum(iy1 - iy0, 0.0)
    pa = (px1 - px0) * (py1 - py0)
    ba = (bx1 - bx0) * (by1 - by0)
    ov = inter / (pa + ba - inter)
    return ov, pcx, pcy, pw, ph, bx0, by0, bx1, by1


def _matchprep_kernel(nP, nPL, nB, nobj,
                      priors_ref, boxes_ref, labels_ref, locsT_ref,
                      code_ref, sm_ref, sp_ref, sd_ref,
                      vacc, iacc, ovx, objs, a_m, a_p, a_d):
    b = pl.program_id(0)
    l = pl.program_id(1)

    @pl.when((b == 0) & (l == 0))
    def _():
        z = jnp.zeros((1, 1), jnp.float32)
        a_m[...] = z
        a_p[...] = z
        a_d[...] = z

    @pl.when(l < nPL)
    def _():
        ov = _iou_lanes(priors_ref, boxes_ref)[0]
        glob = lax.broadcasted_iota(jnp.int32, ov.shape, 1) + l * _L
        ovm = jnp.where(glob < nP, ov, -1.0)
        soi = lax.broadcasted_iota(jnp.int32, ov.shape, 0)
        ovmax_t = jnp.max(ovm, axis=0, keepdims=True)
        obj_t = jnp.min(jnp.where(ovm == ovmax_t, soi, jnp.int32(64)),
                        axis=0, keepdims=True)
        ovx[:, pl.ds(l * _L, _L)] = ovmax_t
        objs[:, pl.ds(l * _L, _L)] = obj_t
        rmax = jnp.max(ovm, axis=1, keepdims=True)
        ridx = jnp.min(jnp.where(ovm == rmax, glob, jnp.int32(2 ** 30)),
                       axis=1, keepdims=True)

        @pl.when(l == 0)
        def _():
            vacc[...] = rmax
            iacc[...] = ridx

        @pl.when(l > 0)
        def _():
            better = rmax > vacc[...]
            iacc[...] = jnp.where(better, ridx, iacc[...])
            vacc[...] = jnp.where(better, rmax, vacc[...])

    @pl.when(l >= nPL)
    def _():
        t2 = l - nPL
        _, pcx, pcy, pw, ph, bx0, by0, bx1, by1 = _iou_lanes(priors_ref,
                                                             boxes_ref)
        ovmax = ovx[:, pl.ds(t2 * _L, _L)]
        obj = objs[:, pl.ds(t2 * _L, _L)]
        lidx = lax.broadcasted_iota(jnp.int32, (1, _L), 1) + t2 * _L
        validp = lidx < nP

        matchm = iacc[...] == lidx
        soi = lax.broadcasted_iota(jnp.int32, matchm.shape, 0)
        mo = jnp.max(jnp.where(matchm, soi, -1), axis=0, keepdims=True)
        hit = mo >= 0
        obj = jnp.where(hit, mo, obj)
        ovmax = jnp.where(hit, 1.0, ovmax)

        onehot = soi == obj
        labf = labels_ref[0].astype(jnp.float32)
        lab = jnp.sum(jnp.where(onehot, labf, 0.0), axis=0, keepdims=True)
        gx0 = jnp.sum(jnp.where(onehot, bx0, 0.0), axis=0, keepdims=True)
        gy0 = jnp.sum(jnp.where(onehot, by0, 0.0), axis=0, keepdims=True)
        gx1 = jnp.sum(jnp.where(onehot, bx1, 0.0), axis=0, keepdims=True)
        gy1 = jnp.sum(jnp.where(onehot, by1, 0.0), axis=0, keepdims=True)

        pos = (ovmax >= _THRESH) & validp
        neg = (ovmax < _THRESH - 0.1) & validp
        msk = pos | neg
        cval = jnp.where(pos, -(lab + _ALPHA),
                         jnp.where(neg, _ALPHA - 1.0, 0.0))
        code_ref[0] = jnp.broadcast_to(cval, (8, _L))

        bcx = (gx0 + gx1) * 0.5
        bcy = (gy0 + gy1) * 0.5
        bw = gx1 - gx0
        bh = gy1 - gy0
        t0 = (bcx - pcx) / (pw * 0.1)
        t1 = (bcy - pcy) / (ph * 0.1)
        t2_ = jnp.log(bw / pw) * 5.0
        t3 = jnp.log(bh / ph) * 5.0
        lt = locsT_ref[0]
        d = (jnp.abs(lt[0:1, :] - t0) + jnp.abs(lt[1:2, :] - t1)
             + jnp.abs(lt[2:3, :] - t2_) + jnp.abs(lt[3:4, :] - t3))

        a_m[...] += jnp.sum(jnp.where(msk, 1.0, 0.0)).reshape(1, 1)
        a_p[...] += jnp.sum(jnp.where(pos, 1.0, 0.0)).reshape(1, 1)
        a_d[...] += jnp.sum(jnp.where(pos, d, 0.0)).reshape(1, 1)

    @pl.when((b == nB - 1) & (l == 2 * nPL - 1))
    def _():
        sm_ref[...] = a_m[...]
        sp_ref[...] = a_p[...]
        sd_ref[...] = a_d[...]


def _loss_kernel(nPT, nB, scores_ref, code_ref,
                 sm_ref, sp_ref, sd_ref, out_ref, a_fl):
    b = pl.program_id(0)
    pt = pl.program_id(1)

    @pl.when((b == 0) & (pt == 0))
    def _():
        a_fl[...] = jnp.zeros((1, 1), jnp.float32)

    x = scores_ref[0]
    c = code_ref[0][0:1, :].reshape(_PT, 1)
    tci = jnp.floor(-c).astype(jnp.int32)
    coef = jnp.where(c < -1.0, -_ALPHA, c)
    mx = jnp.max(x, axis=1, keepdims=True)
    s = x - mx
    lse = jnp.log(jnp.sum(jnp.exp(s), axis=1, keepdims=True))
    cl = lax.broadcasted_iota(jnp.int32, x.shape, 1)
    st = jnp.sum(jnp.where(cl == tci, s, 0.0), axis=1, keepdims=True)
    logpt = st - lse
    om = 1.0 - jnp.exp(logpt)
    f = coef * (om * om) * logpt
    f = jnp.where(c < 0.0, f, 0.0)
    a_fl[...] += jnp.sum(f).reshape(1, 1)

    @pl.when((b == nB - 1) & (pt == nPT - 1))
    def _():
        out_ref[...] = (a_fl[...] / jnp.maximum(sm_ref[...], 1.0)
                        + sd_ref[...] / jnp.maximum(sp_ref[...] * 4.0, 1.0))


def kernel(predicted_locs, predicted_scores, boxes, priors_cxcy, labels):
    B, P, C = predicted_scores.shape
    NOBJ = boxes.shape[1]
    PL = (P + _L - 1) // _L
    Ppad = PL * _L
    PT = Ppad // _PT

    priors_T = priors_cxcy.T
    labels_c = labels[..., None]
    locs_T = predicted_locs.transpose(0, 2, 1)

    code, sm, sp, sd = pl.pallas_call(
        lambda *refs: _matchprep_kernel(P, PL, B, NOBJ, *refs),
        grid=(B, 2 * PL),
        in_specs=[
            pl.BlockSpec((4, _L), lambda b, l: (0, lax.rem(l, PL))),
            pl.BlockSpec((1, NOBJ, 4), lambda b, l: (b, 0, 0)),
            pl.BlockSpec((1, NOBJ, 1), lambda b, l: (b, 0, 0)),
            pl.BlockSpec((1, 4, _L),
                         lambda b, l: (b, 0, jnp.maximum(l - PL, 0))),
        ],
        out_specs=[
            pl.BlockSpec((1, 8, _L),
                         lambda b, l: (b, 0, jnp.maximum(l - PL, 0))),
            pl.BlockSpec((1, 1), lambda b, l: (0, 0)),
            pl.BlockSpec((1, 1), lambda b, l: (0, 0)),
            pl.BlockSpec((1, 1), lambda b, l: (0, 0)),
        ],
        out_shape=[
            jax.ShapeDtypeStruct((B, 8, Ppad), jnp.float32),
            jax.ShapeDtypeStruct((1, 1), jnp.float32),
            jax.ShapeDtypeStruct((1, 1), jnp.float32),
            jax.ShapeDtypeStruct((1, 1), jnp.float32),
        ],
        scratch_shapes=[
            pltpu.VMEM((NOBJ, 1), jnp.float32),
            pltpu.VMEM((NOBJ, 1), jnp.int32),
            pltpu.VMEM((1, Ppad), jnp.float32),
            pltpu.VMEM((1, Ppad), jnp.int32),
            pltpu.VMEM((1, 1), jnp.float32),
            pltpu.VMEM((1, 1), jnp.float32),
            pltpu.VMEM((1, 1), jnp.float32),
        ],
    )(priors_T, boxes, labels_c, locs_T)

    loss = pl.pallas_call(
        lambda *refs: _loss_kernel(PT, B, *refs),
        grid=(B, PT),
        in_specs=[
            pl.BlockSpec((1, _PT, C), lambda b, pt: (b, pt, 0)),
            pl.BlockSpec((1, 8, _PT), lambda b, pt: (b, 0, pt)),
            pl.BlockSpec((1, 1), lambda b, pt: (0, 0)),
            pl.BlockSpec((1, 1), lambda b, pt: (0, 0)),
            pl.BlockSpec((1, 1), lambda b, pt: (0, 0)),
        ],
        out_specs=pl.BlockSpec((1, 1), lambda b, pt: (0, 0)),
        out_shape=jax.ShapeDtypeStruct((1, 1), jnp.float32),
        scratch_shapes=[pltpu.VMEM((1, 1), jnp.float32)],
    )(predicted_scores, code, sm, sp, sd)

    return loss[0, 0]
